# baseline (speedup 1.0000x reference)
import functools

import jax
import jax.numpy as jnp
from jax import lax
from jax.experimental import pallas as pl
from jax.experimental.pallas import tpu as pltpu
from jax.experimental.pallas import tpu_sc as plsc

VOCAB = 100000
EMB = 128
HID = 768
MAXPOS = 4096
B, L = 4, 4096
EPS = 1e-12

N_TOK = B * L
T = 2048
GRID = N_TOK // T
LBLK = L // T

NC, NS = 2, 16
NW = NC * NS
ROWS_PER_W = N_TOK // NW
CHUNK = 128
NCHUNK = ROWS_PER_W // CHUNK
LANES = 16


HALF = ROWS_PER_W // 2


def _sc_gather(table_hbm, idx_hbm, out_hbm, idx_v, rows_v, words_v, sem, sem_w):
    wid = lax.axis_index("s") * NC + lax.axis_index("c")
    blk = wid // 4
    q = wid % 4
    b = blk // (L // T)
    colbase = (blk % (L // T)) * T + q * HALF
    pltpu.sync_copy(idx_hbm.at[b, pl.ds(colbase, HALF)],
                    idx_v.at[pl.ds(0, HALF)])
    pltpu.sync_copy(idx_hbm.at[b, pl.ds(colbase + T // 2, HALF)],
                    idx_v.at[pl.ds(HALF, HALF)])
    gathers = []
    for j in range(NCHUNK):
        cp = pltpu.make_async_copy(
            table_hbm.at[idx_v.at[pl.ds(j * CHUNK, CHUNK)]],
            rows_v.at[pl.ds(j * CHUNK, CHUNK)],
            sem,
        )
        cp.start()
        gathers.append(cp)
    gathers[0].wait()
    gathers[1].wait()
    gathers[2].wait()

    def conv(r, _):
        for k in range(EMB // LANES):
            a = rows_v[r, pl.ds(k * LANES, LANES)]
            bb = rows_v[HALF + r, pl.ds(k * LANES, LANES)]
            lo = lax.shift_right_logical(a + jnp.int32(0x8000), 16)
            hi = (bb + jnp.int32(0x8000)) & jnp.int32(-65536)
            words_v[r, pl.ds(k * LANES, LANES)] = lo | hi
        return 0

    lax.fori_loop(0, CHUNK, conv, 0)
    gathers[3].wait()
    lax.fori_loop(CHUNK, HALF, conv, 0)
    wr = pltpu.make_async_copy(
        words_v, out_hbm.at[pl.ds(wid * HALF, HALF)], sem_w)
    wr.start()
    wr.wait()


@functools.cache
def _gather_words_fn():
    return pl.kernel(
        _sc_gather,
        mesh=plsc.VectorSubcoreMesh(core_axis_name="c", subcore_axis_name="s"),
        out_type=jax.ShapeDtypeStruct((N_TOK // 2, EMB), jnp.int32),
        scratch_types=[
            pltpu.VMEM((ROWS_PER_W,), jnp.int32),
            pltpu.VMEM((ROWS_PER_W, EMB), jnp.int32),
            pltpu.VMEM((ROWS_PER_W // 2, EMB), jnp.int32),
            pltpu.SemaphoreType.DMA,
            pltpu.SemaphoreType.DMA,
        ],
    )


def _tc_body(g_ref, w_ref, pos_ref, tt_ref, par_ref, o_ref):
    i = pl.program_id(0)
    g = g_ref[:, :]
    lo = lax.bitcast_convert_type(g << 16, jnp.float32)
    hi = lax.bitcast_convert_type(g & jnp.int32(-65536), jnp.float32)
    x = jnp.concatenate([lo, hi], axis=0)
    y = jnp.dot(x, w_ref[:, :], preferred_element_type=jnp.float32)
    pos = pos_ref[pl.ds(lax.rem(i, LBLK) * T, T), :]
    t0 = par_ref[0, :]
    t1 = par_ref[1, :]
    gamma = par_ref[2, :]
    beta = par_ref[3, :]
    ttf = tt_ref[0, 0, :].astype(jnp.float32)[:, None]
    y = y + pos + t0[None, :] + ttf * (t1 - t0)[None, :]
    mu = jnp.mean(y, axis=-1, keepdims=True)
    c = y - mu
    var = jnp.mean(c * c, axis=-1, keepdims=True)
    o_ref[:, :] = c * lax.rsqrt(var + EPS) * gamma[None, :] + beta[None, :]


def _tc_call(gathered, W2, pos_emb, tt3, params):
    return pl.pallas_call(
        _tc_body,
        grid=(GRID,),
        in_specs=[
            pl.BlockSpec((T // 2, EMB), lambda i: (i, 0)),
            pl.BlockSpec((EMB, HID), lambda i: (0, 0)),
            pl.BlockSpec((L, HID), lambda i: (0, 0)),
            pl.BlockSpec((1, 1, T), lambda i: (i, 0, 0)),
            pl.BlockSpec((8, HID), lambda i: (0, 0)),
        ],
        out_specs=pl.BlockSpec((T, HID), lambda i: (i, 0)),
        out_shape=jax.ShapeDtypeStruct((N_TOK, HID), jnp.float32),
    )(gathered, W2, pos_emb, tt3, params)


def kernel(input_ids, token_type_ids, word_emb, W2, pos_emb, type_emb, gamma, beta):
    word_bits = lax.bitcast_convert_type(word_emb, jnp.int32)
    gathered = _gather_words_fn()(word_bits, input_ids.astype(jnp.int32))
    tt3 = token_type_ids.reshape(GRID, 1, T).astype(jnp.int32)
    params = jnp.concatenate(
        [type_emb, gamma[None, :], beta[None, :],
         jnp.zeros((4, HID), jnp.float32)], axis=0)
    out = _tc_call(gathered, W2, pos_emb, tt3, params)
    return out.reshape(B, L, HID)


# speedup vs baseline: 1.4651x; 1.4651x over previous
import functools

import jax
import jax.numpy as jnp
from jax import lax
from jax.experimental import pallas as pl
from jax.experimental.pallas import tpu as pltpu
from jax.experimental.pallas import tpu_sc as plsc

VOCAB = 100000
EMB = 128
HID = 768
MAXPOS = 4096
B, L = 4, 4096
EPS = 1e-12

N_TOK = B * L
T = 2048
HALF_L = L // 2
SLAB = B * HALF_L
SGRID = SLAB // T

NC, NS = 2, 16
NW = NC * NS
ROWS_W = SLAB // NW
CHUNK = 128
NCHUNK = ROWS_W // CHUNK


def _sc_gather(table_hbm, idx_hbm, out_hbm, idx_v, rows_v, sem, sem_w):
    wid = lax.axis_index("s") * NC + lax.axis_index("c")
    w_per_row = HALF_L // ROWS_W
    b = wid // w_per_row
    col = (wid % w_per_row) * ROWS_W
    pltpu.sync_copy(idx_hbm.at[b, pl.ds(col, ROWS_W)], idx_v)
    gathers = []
    for j in range(NCHUNK):
        cp = pltpu.make_async_copy(
            table_hbm.at[idx_v.at[pl.ds(j * CHUNK, CHUNK)]],
            rows_v.at[pl.ds(j * CHUNK, CHUNK)],
            sem,
        )
        cp.start()
        gathers.append(cp)
    writes = []
    for j in range(NCHUNK):
        gathers[j].wait()
        wr = pltpu.make_async_copy(
            rows_v.at[pl.ds(j * CHUNK, CHUNK)],
            out_hbm.at[pl.ds(wid * ROWS_W + j * CHUNK, CHUNK)],
            sem_w,
        )
        wr.start()
        writes.append(wr)
    for wr in writes:
        wr.wait()


@functools.cache
def _gather_words_fn():
    return pl.kernel(
        _sc_gather,
        mesh=plsc.VectorSubcoreMesh(core_axis_name="c", subcore_axis_name="s"),
        out_type=jax.ShapeDtypeStruct((SLAB, EMB), jnp.float32),
        scratch_types=[
            pltpu.VMEM((ROWS_W,), jnp.int32),
            pltpu.VMEM((ROWS_W, EMB), jnp.float32),
            pltpu.SemaphoreType.DMA,
            pltpu.SemaphoreType.DMA,
        ],
    )


def _tc_body(g_ref, w_ref, pos_ref, tt_ref, par_ref, o_ref):
    y = jnp.dot(g_ref[:, :], w_ref[:, :], preferred_element_type=jnp.float32)
    t0 = par_ref[0, :]
    t1 = par_ref[1, :]
    gamma = par_ref[2, :]
    beta = par_ref[3, :]
    ttf = tt_ref[0, 0, :].astype(jnp.float32)[:, None]
    y = y + pos_ref[:, :] + t0[None, :] + ttf * (t1 - t0)[None, :]
    mu = jnp.mean(y, axis=-1, keepdims=True)
    c = y - mu
    var = jnp.mean(c * c, axis=-1, keepdims=True)
    o_ref[:, :] = c * lax.rsqrt(var + EPS) * gamma[None, :] + beta[None, :]


def _tc_body_alias(g_ref, w_ref, pos_ref, tt_ref, par_ref, prev_ref, o_ref):
    _tc_body(g_ref, w_ref, pos_ref, tt_ref, par_ref, o_ref)


def _tc_call(s, gathered, W2, pos_emb, tt3, params, prev=None):
    in_specs = [
        pl.BlockSpec((T, EMB), lambda i: (i, 0)),
        pl.BlockSpec((EMB, HID), lambda i: (0, 0)),
        pl.BlockSpec((T, HID), lambda i, s=s: (s, 0)),
        pl.BlockSpec((1, 1, T), lambda i, s=s: (2 * i + s, 0, 0)),
        pl.BlockSpec((8, HID), lambda i: (0, 0)),
    ]
    args = [gathered, W2, pos_emb, tt3, params]
    body = _tc_body
    kwargs = {}
    if prev is not None:
        in_specs.append(pl.BlockSpec(memory_space=pl.ANY))
        args.append(prev)
        body = _tc_body_alias
        kwargs["input_output_aliases"] = {5: 0}
    return pl.pallas_call(
        body,
        grid=(SGRID,),
        in_specs=in_specs,
        out_specs=pl.BlockSpec((T, HID), lambda i, s=s: (2 * i + s, 0)),
        out_shape=jax.ShapeDtypeStruct((N_TOK, HID), jnp.float32),
        **kwargs,
    )(*args)


def kernel(input_ids, token_type_ids, word_emb, W2, pos_emb, type_emb, gamma, beta):
    ids = input_ids.astype(jnp.int32)
    g0 = _gather_words_fn()(word_emb, ids[:, :HALF_L])
    g1 = _gather_words_fn()(word_emb, ids[:, HALF_L:])
    tt3 = token_type_ids.reshape(N_TOK // T, 1, T).astype(jnp.int32)
    params = jnp.concatenate(
        [type_emb, gamma[None, :], beta[None, :],
         jnp.zeros((4, HID), jnp.float32)], axis=0)
    o0 = _tc_call(0, g0, W2, pos_emb, tt3, params)
    out = _tc_call(1, g1, W2, pos_emb, tt3, params, prev=o0)
    return out.reshape(B, L, HID)


# speedup vs baseline: 1.5238x; 1.0400x over previous
import functools

import jax
import jax.numpy as jnp
from jax import lax
from jax.experimental import pallas as pl
from jax.experimental.pallas import tpu as pltpu
from jax.experimental.pallas import tpu_sc as plsc

VOCAB = 100000
EMB = 128
HID = 768
MAXPOS = 4096
B, L = 4, 4096
EPS = 1e-12

N_TOK = B * L
T = 2048
GRID = N_TOK // T
LBLK = L // T

NC, NS = 2, 16
NW = NC * NS
ROWS_PER_W = N_TOK // NW
CHUNK = 128
NCHUNK = ROWS_PER_W // CHUNK


def _sc_gather(table_hbm, idx_hbm, out_hbm, idx_v, rows_v, sem, sem_w):
    wid = lax.axis_index("s") * NC + lax.axis_index("c")
    w_per_row = L // ROWS_PER_W
    b = wid // w_per_row
    col = (wid % w_per_row) * ROWS_PER_W
    pltpu.sync_copy(idx_hbm.at[b, pl.ds(col, ROWS_PER_W)], idx_v)
    gathers = []
    for j in range(NCHUNK):
        cp = pltpu.make_async_copy(
            table_hbm.at[idx_v.at[pl.ds(j * CHUNK, CHUNK)]],
            rows_v.at[pl.ds(j * CHUNK, CHUNK)],
            sem,
        )
        cp.start()
        gathers.append(cp)
    writes = []
    for j in range(NCHUNK):
        gathers[j].wait()
        wr = pltpu.make_async_copy(
            rows_v.at[pl.ds(j * CHUNK, CHUNK)],
            out_hbm.at[pl.ds(wid * ROWS_PER_W + j * CHUNK, CHUNK)],
            sem_w,
        )
        wr.start()
        writes.append(wr)
    for wr in writes:
        wr.wait()


@functools.cache
def _gather_words_fn():
    return pl.kernel(
        _sc_gather,
        mesh=plsc.VectorSubcoreMesh(core_axis_name="c", subcore_axis_name="s"),
        out_type=jax.ShapeDtypeStruct((N_TOK, EMB), jnp.float32),
        scratch_types=[
            pltpu.VMEM((ROWS_PER_W,), jnp.int32),
            pltpu.VMEM((ROWS_PER_W, EMB), jnp.float32),
            pltpu.SemaphoreType.DMA,
            pltpu.SemaphoreType.DMA,
        ],
    )


def _tc_body(g_ref, w_ref, pos_ref, tt_ref, par_ref, o_ref):
    i = pl.program_id(0)
    y = jnp.dot(g_ref[:, :], w_ref[:, :], preferred_element_type=jnp.float32)
    pos = pos_ref[pl.ds(lax.rem(i, LBLK) * T, T), :]
    t0 = par_ref[0, :]
    t1 = par_ref[1, :]
    gamma = par_ref[2, :]
    beta = par_ref[3, :]
    ttf = tt_ref[0, 0, :].astype(jnp.float32)[:, None]
    y = y + pos + t0[None, :] + ttf * (t1 - t0)[None, :]
    mu = jnp.mean(y, axis=-1, keepdims=True)
    c = y - mu
    var = jnp.mean(c * c, axis=-1, keepdims=True)
    o_ref[:, :] = c * lax.rsqrt(var + EPS) * gamma[None, :] + beta[None, :]


def _tc_call(gathered, W2, pos_emb, tt3, params):
    return pl.pallas_call(
        _tc_body,
        grid=(GRID,),
        in_specs=[
            pl.BlockSpec((T, EMB), lambda i: (i, 0)),
            pl.BlockSpec((EMB, HID), lambda i: (0, 0)),
            pl.BlockSpec((L, HID), lambda i: (0, 0)),
            pl.BlockSpec((1, 1, T), lambda i: (i, 0, 0)),
            pl.BlockSpec((8, HID), lambda i: (0, 0)),
        ],
        out_specs=pl.BlockSpec((T, HID), lambda i: (i, 0)),
        out_shape=jax.ShapeDtypeStruct((N_TOK, HID), jnp.float32),
    )(gathered, W2, pos_emb, tt3, params)


def kernel(input_ids, token_type_ids, word_emb, W2, pos_emb, type_emb, gamma, beta):
    gathered = _gather_words_fn()(word_emb, input_ids.astype(jnp.int32))
    tt3 = token_type_ids.reshape(GRID, 1, T).astype(jnp.int32)
    params = jnp.concatenate(
        [type_emb, gamma[None, :], beta[None, :],
         jnp.zeros((4, HID), jnp.float32)], axis=0)
    out = _tc_call(gathered, W2, pos_emb, tt3, params)
    return out.reshape(B, L, HID)
